# baseline (speedup 1.0000x reference)
import numpy as np
import jax
import jax.numpy as jnp
from jax import lax
from jax.experimental import pallas as pl
from jax.experimental.pallas import tpu as pltpu

_NUM_CLASSES = 80
_ATTRIB = 85
_STRIDES = (32, 16, 8)
_BASE = (((116, 90), (156, 198), (373, 326)),
         ((30, 61), (62, 45), (59, 119)),
         ((10, 13), (16, 30), (33, 23)))
_HW = (19, 38, 76)
_CONF_THR = 0.005
_NMS_PRE = 1000
_IOU_THR = 0.45
_MAX_PER_IMG = 100
_BN_EPS = 1e-5
_LEAKY = 0.1
_PREC = lax.Precision.DEFAULT


def _rup(x, m):
    return (x + m - 1) // m * m


def _head_body(x_ref, w_ref, bn_ref, wp_ref, bpp_ref, anc_ref, base_ref,
               attr_ref, o_ref, *, n_blk, mb, wp1, stride):
    scale = bn_ref[0:1, :] * lax.rsqrt(bn_ref[3:4, :] + _BN_EPS)
    beta = bn_ref[1:2, :]
    mean = bn_ref[2:3, :]
    attr = attr_ref[0:1, :]
    anc = anc_ref[0:1, :]
    for blk in range(n_blk):
        r0 = blk * mb
        acc = None
        for dy in range(3):
            for dx in (-1, 0, 1):
                s = r0 + dy * wp1 + dx + 1
                t = dy * 3 + dx + 1
                c = jnp.dot(x_ref[0, s:s + mb, :], w_ref[t],
                            preferred_element_type=jnp.float32,
                            precision=_PREC)
                acc = c if acc is None else acc + c
        y = (acc - mean) * scale + beta
        y = jnp.where(y >= 0, y, _LEAKY * y)
        pm = jnp.dot(y, wp_ref[...], preferred_element_type=jnp.float32,
                     precision=_PREC) + bpp_ref[0:1, :]
        sig = 1.0 / (1.0 + jnp.exp(-pm))
        ex = jnp.exp(pm)
        base = base_ref[r0:r0 + mb, :]
        ctr = base + (sig - 0.5) * stride
        whh = anc * ex
        tl = ctr - pltpu.roll(whh, 254, axis=1)
        br = pltpu.roll(ctr, 2, axis=1) + whh
        dec = jnp.where(attr < 2.0, tl, jnp.where(attr < 4.0, br, sig))
        o_ref[0, r0:r0 + mb, :] = dec


def _head_level_pallas(feat, wb, gamma, beta, mean, var, wp, bp, idx, mb):
    b, cin, h, w0 = feat.shape
    cout = wb.shape[0]
    stride = _STRIDES[idx]
    wp1 = w0 + 1
    rp = h * wp1
    rp8 = _rup(rp, 8)
    n_blk = rp8 // mb
    lpad = _rup(rp8 + 2 * wp1 + 2, 8)

    xn = feat.transpose(0, 2, 3, 1)
    xp = jnp.pad(xn, ((0, 0), (1, 1), (0, 1), (0, 0)))
    xf = xp.reshape(b, (h + 2) * wp1, cin)
    xf = jnp.pad(xf, ((0, 0), (1, lpad - 1 - (h + 2) * wp1), (0, 0)))

    wt = wb.transpose(2, 3, 1, 0).reshape(9, cin, cout)
    bn = jnp.stack([gamma, beta, mean, var], 0)
    wpt = jnp.pad(wp.T, ((0, 0), (0, 1)))
    bpp = jnp.pad(bp, (0, 1)).reshape(1, 256)

    lane = np.arange(256)
    attr_np = (lane % _ATTRIB).astype(np.float32)
    attr_np[255] = 84.0
    a_np = np.minimum(lane // _ATTRIB, 2)
    ws = np.array([s[0] for s in _BASE[idx]], np.float32)
    hs = np.array([s[1] for s in _BASE[idx]], np.float32)
    anc_np = np.where(attr_np == 2.0, 0.5 * ws[a_np],
                      np.where(attr_np == 3.0, 0.5 * hs[a_np], 0.0))
    p = np.arange(rp8)
    hh = np.minimum(p // wp1, h - 1).astype(np.float32)
    ww = (p % wp1).astype(np.float32)
    base_np = np.zeros((rp8, 256), np.float32)
    base_np[:, :] = np.where(attr_np[None, :] == 0.0,
                             (ww[:, None] + 0.5) * stride,
                             np.where(attr_np[None, :] == 1.0,
                                      (hh[:, None] + 0.5) * stride, 0.0))

    out = pl.pallas_call(
        lambda *refs: _head_body(*refs, n_blk=n_blk, mb=mb, wp1=wp1,
                                 stride=float(stride)),
        grid=(b,),
        in_specs=[
            pl.BlockSpec((1, lpad, cin), lambda i: (i, 0, 0)),
            pl.BlockSpec((9, cin, cout), lambda i: (0, 0, 0)),
            pl.BlockSpec((4, cout), lambda i: (0, 0)),
            pl.BlockSpec((cout, 256), lambda i: (0, 0)),
            pl.BlockSpec((1, 256), lambda i: (0, 0)),
            pl.BlockSpec((1, 256), lambda i: (0, 0)),
            pl.BlockSpec((rp8, 256), lambda i: (0, 0)),
            pl.BlockSpec((1, 256), lambda i: (0, 0)),
        ],
        out_specs=pl.BlockSpec((1, rp8, 256), lambda i: (i, 0, 0)),
        out_shape=jax.ShapeDtypeStruct((b, rp8, 256), jnp.float32),
        compiler_params=pltpu.CompilerParams(
            dimension_semantics=("parallel",),
            vmem_limit_bytes=56 * 1024 * 1024,
        ),
        name=f"yolo_head_l{idx}",
    )(xf, wt, bn, wpt, bpp,
      jnp.asarray(anc_np).reshape(1, 256), jnp.asarray(base_np),
      jnp.asarray(attr_np).reshape(1, 256))

    dec = out[:, :rp, :255].reshape(b, h, wp1, 3, 85)[:, :, :w0]
    return dec.reshape(b, h * w0 * 3, 85)


def _iou_mat(a):
    area = (a[:, 2] - a[:, 0]) * (a[:, 3] - a[:, 1])
    lt = jnp.maximum(a[:, None, :2], a[None, :, :2])
    rb = jnp.minimum(a[:, None, 2:], a[None, :, 2:])
    wh = jnp.clip(rb - lt, 0.0)
    inter = wh[..., 0] * wh[..., 1]
    return inter / (area[:, None] + area[None, :] - inter + 1e-6)


def _detect_one(boxes, conf, cls):
    conf = jnp.where(conf >= _CONF_THR, conf, 0.0)
    conf_k, idx = lax.top_k(conf, _NMS_PRE)
    boxes_k = boxes[idx]
    cls_k = cls[idx]
    score = conf_k * jnp.max(cls_k, -1)
    label = jnp.argmax(cls_k, -1)
    order = jnp.argsort(-score)
    boxes_s = boxes_k[order]
    score_s = score[order]
    label_s = label[order]
    iou = _iou_mat(boxes_s)
    rng = jnp.arange(_NMS_PRE)

    def body(i, keep):
        sup = keep[i] & (iou[i] > _IOU_THR) & (rng > i)
        return keep & ~sup

    keep = lax.fori_loop(0, _NMS_PRE, body, score_s > 0)
    final = jnp.where(keep, score_s, 0.0)
    top_s, top_i = lax.top_k(final, _MAX_PER_IMG)
    det = jnp.concatenate([boxes_s[top_i], top_s[:, None]], -1)
    return det, label_s[top_i]


def kernel(feat0, feat1, feat2,
           wb0, gamma0, beta0, mean0, var0, wp0, bp0,
           wb1, gamma1, beta1, mean1, var1, wp1, bp1,
           wb2, gamma2, beta2, mean2, var2, wp2, bp2):
    feats = (feat0, feat1, feat2)
    params = ((wb0, gamma0, beta0, mean0, var0, wp0, bp0),
              (wb1, gamma1, beta1, mean1, var1, wp1, bp1),
              (wb2, gamma2, beta2, mean2, var2, wp2, bp2))
    mbs = (384, 744, 1464)
    outs = []
    for i in range(3):
        outs.append(_head_level_pallas(feats[i], *params[i], i, mbs[i]))
    allp = jnp.concatenate(outs, 1)
    boxes = allp[..., :4]
    conf = allp[..., 4]
    cls = allp[..., 5:]
    det, labels = jax.vmap(_detect_one)(boxes, conf, cls)
    return det, labels


# speedup vs baseline: 1.1052x; 1.1052x over previous
import numpy as np
import jax
import jax.numpy as jnp
from jax import lax
from jax.experimental import pallas as pl
from jax.experimental.pallas import tpu as pltpu

_NUM_CLASSES = 80
_ATTRIB = 85
_STRIDES = (32, 16, 8)
_BASE = (((116, 90), (156, 198), (373, 326)),
         ((30, 61), (62, 45), (59, 119)),
         ((10, 13), (16, 30), (33, 23)))
_HW = (19, 38, 76)
_CONF_THR = 0.005
_NMS_PRE = 1000
_IOU_THR = 0.45
_MAX_PER_IMG = 100
_BN_EPS = 1e-5
_LEAKY = 0.1
_PREC = lax.Precision.DEFAULT


def _rup(x, m):
    return (x + m - 1) // m * m


def _head_body(x_ref, w_ref, bn_ref, wp_ref, bpp_ref, anc_ref, base_ref,
               attr_ref, o_ref, *, n_blk, mb, wp1, stride, h, w0):
    scale = bn_ref[0:1, :] * lax.rsqrt(bn_ref[3:4, :] + _BN_EPS)
    beta = bn_ref[1:2, :]
    mean = bn_ref[2:3, :]
    attr = attr_ref[0:1, :]
    anc = anc_ref[0:1, :]
    lane = lax.broadcasted_iota(jnp.int32, (1, 256), 1)
    clsmask = (attr >= 5.0) & (attr < 85.0)
    for blk in range(n_blk):
        r0 = blk * mb
        acc = None
        for dy in range(3):
            for dx in (-1, 0, 1):
                s = r0 + dy * wp1 + dx + 1
                t = dy * 3 + dx + 1
                c = jnp.dot(x_ref[0, s:s + mb, :], w_ref[t],
                            preferred_element_type=jnp.float32,
                            precision=_PREC)
                acc = c if acc is None else acc + c
        y = (acc - mean) * scale + beta
        y = jnp.where(y >= 0, y, _LEAKY * y)
        pm = jnp.dot(y, wp_ref[...], preferred_element_type=jnp.float32,
                     precision=_PREC) + bpp_ref[0:1, :]
        sig = 1.0 / (1.0 + jnp.exp(-pm))
        ex = jnp.exp(pm)
        base = base_ref[r0:r0 + mb, :]
        ctr = base + (sig - 0.5) * stride
        whh = anc * ex
        tl = ctr - pltpu.roll(whh, 254, axis=1)
        br = pltpu.roll(ctr, 2, axis=1) + whh

        pidx = lax.broadcasted_iota(jnp.int32, (mb, 256), 0) + r0
        wwi = pidx - (pidx // wp1) * wp1
        valid = (wwi != w0) & (pidx < h * wp1)
        confv = jnp.where(valid, sig, 0.0)

        v = jnp.where(clsmask, sig, -1.0)
        iv = jnp.where(clsmask, attr - 5.0, 1000.0)
        for k in (1, 2, 4, 8, 16, 32, 16):
            v2 = pltpu.roll(v, 256 - k, axis=1)
            i2 = pltpu.roll(iv, 256 - k, axis=1)
            take = (v2 > v) | ((v2 == v) & (i2 < iv))
            v = jnp.where(take, v2, v)
            iv = jnp.where(take, i2, iv)
        score5 = v * pltpu.roll(confv, 1, axis=1)
        label6 = pltpu.roll(iv, 1, axis=1)

        outv = jnp.where(attr < 2.0, tl,
               jnp.where(attr < 4.0, br,
               jnp.where(attr == 4.0, confv,
               jnp.where(attr == 5.0, score5,
               jnp.where(attr == 6.0, label6, 0.0)))))
        r1 = pltpu.roll(outv, 179, axis=1)
        r2 = pltpu.roll(outv, 102, axis=1)
        comp = jnp.where(lane < 8, outv, jnp.where(lane < 16, r1, r2))
        o_ref[0, r0:r0 + mb, :] = comp[:, :128]


def _head_level_pallas(feat, wb, gamma, beta, mean, var, wp, bp, idx, mb):
    b, cin, h, w0 = feat.shape
    cout = wb.shape[0]
    stride = _STRIDES[idx]
    wp1 = w0 + 1
    rp = h * wp1
    rp8 = _rup(rp, 8)
    n_blk = rp8 // mb
    lpad = _rup(rp8 + 2 * wp1 + 2, 8)

    xn = feat.transpose(0, 2, 3, 1)
    xp = jnp.pad(xn, ((0, 0), (1, 1), (0, 1), (0, 0)))
    xf = xp.reshape(b, (h + 2) * wp1, cin)
    xf = jnp.pad(xf, ((0, 0), (1, lpad - 1 - (h + 2) * wp1), (0, 0)))

    wt = wb.transpose(2, 3, 1, 0).reshape(9, cin, cout)
    bn = jnp.stack([gamma, beta, mean, var], 0)
    wpt = jnp.pad(wp.T, ((0, 0), (0, 1)))
    bpp = jnp.pad(bp, (0, 1)).reshape(1, 256)

    lane = np.arange(256)
    attr_np = (lane % _ATTRIB).astype(np.float32)
    attr_np[255] = 100.0
    a_np = np.minimum(lane // _ATTRIB, 2)
    ws = np.array([s[0] for s in _BASE[idx]], np.float32)
    hs = np.array([s[1] for s in _BASE[idx]], np.float32)
    anc_np = np.where(attr_np == 2.0, 0.5 * ws[a_np],
                      np.where(attr_np == 3.0, 0.5 * hs[a_np], 0.0))
    p = np.arange(rp8)
    hh = np.minimum(p // wp1, h - 1).astype(np.float32)
    ww = (p % wp1).astype(np.float32)
    base_np = np.zeros((rp8, 256), np.float32)
    base_np[:, :] = np.where(attr_np[None, :] == 0.0,
                             (ww[:, None] + 0.5) * stride,
                             np.where(attr_np[None, :] == 1.0,
                                      (hh[:, None] + 0.5) * stride, 0.0))

    out = pl.pallas_call(
        lambda *refs: _head_body(*refs, n_blk=n_blk, mb=mb, wp1=wp1,
                                 stride=float(stride), h=h, w0=w0),
        grid=(b,),
        in_specs=[
            pl.BlockSpec((1, lpad, cin), lambda i: (i, 0, 0)),
            pl.BlockSpec((9, cin, cout), lambda i: (0, 0, 0)),
            pl.BlockSpec((4, cout), lambda i: (0, 0)),
            pl.BlockSpec((cout, 256), lambda i: (0, 0)),
            pl.BlockSpec((1, 256), lambda i: (0, 0)),
            pl.BlockSpec((1, 256), lambda i: (0, 0)),
            pl.BlockSpec((rp8, 256), lambda i: (0, 0)),
            pl.BlockSpec((1, 256), lambda i: (0, 0)),
        ],
        out_specs=pl.BlockSpec((1, rp8, 128), lambda i: (i, 0, 0)),
        out_shape=jax.ShapeDtypeStruct((b, rp8, 128), jnp.float32),
        compiler_params=pltpu.CompilerParams(
            dimension_semantics=("parallel",),
            vmem_limit_bytes=56 * 1024 * 1024,
        ),
        name=f"yolo_head_l{idx}",
    )(xf, wt, bn, wpt, bpp,
      jnp.asarray(anc_np).reshape(1, 256), jnp.asarray(base_np),
      jnp.asarray(attr_np).reshape(1, 256))

    return out[:, :, :24].reshape(b, rp8 * 3, 8)


def _iou_mat(a):
    area = (a[:, 2] - a[:, 0]) * (a[:, 3] - a[:, 1])
    lt = jnp.maximum(a[:, None, :2], a[None, :, :2])
    rb = jnp.minimum(a[:, None, 2:], a[None, :, 2:])
    wh = jnp.clip(rb - lt, 0.0)
    inter = wh[..., 0] * wh[..., 1]
    return inter / (area[:, None] + area[None, :] - inter + 1e-6)


def _detect_one(arr):
    conf = arr[:, 4]
    conf = jnp.where(conf >= _CONF_THR, conf, 0.0)
    conf_k, idx = lax.top_k(conf, _NMS_PRE)
    g = arr[idx]
    score = jnp.where(conf_k > 0, g[:, 5], 0.0)
    label = g[:, 6].astype(jnp.int32)
    order = jnp.argsort(-score)
    g_s = g[order]
    score_s = score[order]
    label_s = label[order]
    boxes_s = g_s[:, :4]
    iou = _iou_mat(boxes_s)
    rng = jnp.arange(_NMS_PRE)

    def body(i, keep):
        sup = keep[i] & (iou[i] > _IOU_THR) & (rng > i)
        return keep & ~sup

    keep = lax.fori_loop(0, _NMS_PRE, body, score_s > 0)
    final = jnp.where(keep, score_s, 0.0)
    top_s, top_i = lax.top_k(final, _MAX_PER_IMG)
    det = jnp.concatenate([boxes_s[top_i], top_s[:, None]], -1)
    return det, label_s[top_i]


def kernel(feat0, feat1, feat2,
           wb0, gamma0, beta0, mean0, var0, wp0, bp0,
           wb1, gamma1, beta1, mean1, var1, wp1, bp1,
           wb2, gamma2, beta2, mean2, var2, wp2, bp2):
    feats = (feat0, feat1, feat2)
    params = ((wb0, gamma0, beta0, mean0, var0, wp0, bp0),
              (wb1, gamma1, beta1, mean1, var1, wp1, bp1),
              (wb2, gamma2, beta2, mean2, var2, wp2, bp2))
    mbs = (384, 744, 1464)
    outs = []
    for i in range(3):
        outs.append(_head_level_pallas(feats[i], *params[i], i, mbs[i]))
    allp = jnp.concatenate(outs, 1)
    det, labels = jax.vmap(_detect_one)(allp)
    return det, labels
